# Initial kernel scaffold; baseline (speedup 1.0000x reference)
import jax
import jax.numpy as jnp
from jax.experimental import pallas as pl


def kernel(ment_starts, ment_ends, ment_scores, k):
    raise NotImplementedError("write your pallas kernel here")


# speedup vs baseline: 126.1653x; 126.1653x over previous
import jax
import jax.numpy as jnp
from jax import lax
from jax.experimental import pallas as pl
from jax.experimental.pallas import tpu as pltpu
from jax.experimental.pallas import tpu_sc as plsc

_N = 20000
_K = 4000
_TAB = 100064


def _greedy_body(packed_hbm, ztab_hbm, fill_hbm, kvec_hbm, out_hbm,
                 packed_v, table_v, top_v, kv):
    cid = lax.axis_index("c")
    sid = lax.axis_index("s")

    @pl.when(jnp.logical_and(cid == 0, sid == 0))
    def _():
        pltpu.sync_copy(packed_hbm, packed_v)
        pltpu.sync_copy(ztab_hbm, table_v)
        pltpu.sync_copy(fill_hbm, top_v)
        pltpu.sync_copy(kvec_hbm, kv)
        kk = kv[...][0]
        lanes = lax.iota(jnp.int32, 16)
        d1 = lanes + 16

        def step(t, cnt):
            sw = plsc.load_gather(packed_v, [jnp.full((16,), t, jnp.int32)])[0]
            s = sw >> 5
            w1 = sw & 31
            lim = w1 + 1
            idx0 = s + lanes
            v0 = plsc.load_gather(table_v, [idx0])
            v1 = plsc.load_gather(table_v, [idx0 + 16])
            a0 = v0 >> 5
            b0 = v0 & 31
            a1 = v1 >> 5
            b1 = v1 & 31
            bad0 = ((lanes >= 1) & (lanes <= w1) & (a0 > lim - lanes)) | \
                   ((lanes < w1) & (b0 > lanes + 1))
            bad1 = ((d1 <= w1) & (a1 > lim - d1)) | \
                   ((d1 < w1) & (b1 > d1 + 1))
            ok = jnp.logical_and(jnp.logical_not(jnp.any(bad0 | bad1)),
                                 cnt < kk)

            e = s + w1
            ts = v0[0]
            te = plsc.load_gather(
                table_v, [jnp.full((16,), e, jnp.int32)])[0]
            addb = jnp.where(s == e, lim, jnp.int32(0))
            news = (jnp.maximum(ts >> 5, lim) << 5) | \
                jnp.maximum(ts & 31, addb)
            newe = (jnp.maximum(te >> 5, addb) << 5) | \
                jnp.maximum(te & 31, lim)
            idxv = jnp.where(lanes == 0, s, e)
            valv = jnp.where(lanes == 0, news, newe)
            plsc.store_scatter(table_v, [idxv], valv,
                               mask=(lanes < 2) & ok)
            plsc.store_scatter(
                top_v, [jnp.full((16,), cnt, jnp.int32)],
                jnp.full((16,), t, jnp.int32), mask=(lanes == 0) & ok)

            return cnt + jnp.where(ok, jnp.int32(1), jnp.int32(0))

        lax.fori_loop(0, _N, step, jnp.int32(0))
        pltpu.sync_copy(top_v, out_hbm)


def kernel(ment_starts, ment_ends, ment_scores, k):
    starts = ment_starts.astype(jnp.int32)
    ends = ment_ends.astype(jnp.int32)
    scores = jnp.asarray(ment_scores)
    order = jnp.argsort(-scores, stable=True).astype(jnp.int32)
    ssort = starts[order]
    wsort = ends[order] - ssort
    packed = ssort * 32 + wsort

    ztab = jnp.zeros((_TAB,), jnp.int32)
    fill = jnp.full((_K,), -1, jnp.int32)
    kvec = jnp.full((16,), jnp.asarray(k, jnp.int32))

    mesh = plsc.VectorSubcoreMesh(core_axis_name="c", subcore_axis_name="s",
                                  num_cores=2, num_subcores=16)
    ranks = pl.kernel(
        _greedy_body,
        out_type=jax.ShapeDtypeStruct((_K,), jnp.int32),
        mesh=mesh,
        scratch_types=[
            pltpu.VMEM((_N,), jnp.int32),
            pltpu.VMEM((_TAB,), jnp.int32),
            pltpu.VMEM((_K,), jnp.int32),
            pltpu.VMEM((16,), jnp.int32),
        ],
        compiler_params=pltpu.CompilerParams(needs_layout_passes=False),
    )(packed, ztab, fill, kvec)

    valid_sel = ranks >= 0
    top = jnp.where(valid_sel, order[jnp.where(valid_sel, ranks, 0)], -1)

    big = jnp.int32(1 << 30)
    safe_top = jnp.where(valid_sel, top, 0)
    s_sel = starts[safe_top]
    e_sel = ends[safe_top]
    m_last = ends[-1]
    a = (s_sel // 256) * m_last
    b = (s_sel % 256) * m_last + e_sel
    t_lo = (a % 256) * 256 + b
    lo = t_lo % 65536
    hi = (a // 256) + t_lo // 65536
    hi = jnp.where(valid_sel, hi, big)
    lo = jnp.where(valid_sel, lo, jnp.int32(0))
    _, _, idx = lax.sort((hi, lo, top), num_keys=2, is_stable=True)

    valid = idx >= 0
    safe = jnp.where(valid, idx, 0)
    sel_scores = jnp.where(valid, jnp.take(scores, safe), 0.0)
    return (idx, sel_scores)


# speedup vs baseline: 362.3169x; 2.8718x over previous
import jax
import jax.numpy as jnp
from jax import lax
from jax.experimental import pallas as pl
from jax.experimental.pallas import tpu as pltpu
from jax.experimental.pallas import tpu_sc as plsc

_N = 20000
_K = 4000
_CH = 250
_TAB = 100064


def _greedy_body(packed_hbm, ztab_hbm, fill_hbm, kvec_hbm, out_hbm,
                 packed_v, table_v, top_v, kv):
    cid = lax.axis_index("c")
    sid = lax.axis_index("s")

    @pl.when(jnp.logical_and(cid == 0, sid == 0))
    def _():
        pltpu.sync_copy(packed_hbm, packed_v)
        pltpu.sync_copy(ztab_hbm, table_v)
        pltpu.sync_copy(fill_hbm, top_v)
        pltpu.sync_copy(kvec_hbm, kv)
        kk = kv[...][0]
        lanes = lax.iota(jnp.int32, 16)
        d1 = lanes + 16

        def step(t, cnt):
            sw = plsc.load_gather(packed_v, [jnp.full((16,), t, jnp.int32)])[0]
            s = sw >> 5
            w1 = sw & 31
            lim = w1 + 1
            idx0 = s + lanes
            v0 = plsc.load_gather(table_v, [idx0])
            v1 = plsc.load_gather(table_v, [idx0 + 16])
            a0 = v0 >> 5
            b0 = v0 & 31
            a1 = v1 >> 5
            b1 = v1 & 31
            bad0 = ((lanes >= 1) & (lanes <= w1) & (a0 > lim - lanes)) | \
                   ((lanes < w1) & (b0 > lanes + 1))
            bad1 = ((d1 <= w1) & (a1 > lim - d1)) | \
                   ((d1 < w1) & (b1 > d1 + 1))
            ok = jnp.logical_and(jnp.logical_not(jnp.any(bad0 | bad1)),
                                 cnt < kk)

            e = s + w1
            ts = v0[0]
            te = plsc.load_gather(
                table_v, [jnp.full((16,), e, jnp.int32)])[0]
            addb = jnp.where(s == e, lim, jnp.int32(0))
            news = (jnp.maximum(ts >> 5, lim) << 5) | \
                jnp.maximum(ts & 31, addb)
            newe = (jnp.maximum(te >> 5, addb) << 5) | \
                jnp.maximum(te & 31, lim)
            idxv = jnp.where(lanes == 0, s, e)
            valv = jnp.where(lanes == 0, news, newe)
            plsc.store_scatter(table_v, [idxv], valv,
                               mask=(lanes < 2) & ok)
            plsc.store_scatter(
                top_v, [jnp.full((16,), cnt, jnp.int32)],
                jnp.full((16,), t, jnp.int32), mask=(lanes == 0) & ok)

            return cnt + jnp.where(ok, jnp.int32(1), jnp.int32(0))

        def chunk(ci, cnt):
            return lax.cond(
                cnt < kk,
                lambda c: lax.fori_loop(ci * _CH, (ci + 1) * _CH, step, c),
                lambda c: c,
                cnt)

        lax.fori_loop(0, _N // _CH, chunk, jnp.int32(0))
        pltpu.sync_copy(top_v, out_hbm)


def kernel(ment_starts, ment_ends, ment_scores, k):
    starts = ment_starts.astype(jnp.int32)
    ends = ment_ends.astype(jnp.int32)
    scores = jnp.asarray(ment_scores)
    order = jnp.argsort(-scores, stable=True).astype(jnp.int32)
    ssort = starts[order]
    wsort = ends[order] - ssort
    packed = ssort * 32 + wsort

    ztab = jnp.zeros((_TAB,), jnp.int32)
    fill = jnp.full((_K,), -1, jnp.int32)
    kvec = jnp.full((16,), jnp.asarray(k, jnp.int32))

    mesh = plsc.VectorSubcoreMesh(core_axis_name="c", subcore_axis_name="s",
                                  num_cores=2, num_subcores=16)
    ranks = pl.kernel(
        _greedy_body,
        out_type=jax.ShapeDtypeStruct((_K,), jnp.int32),
        mesh=mesh,
        scratch_types=[
            pltpu.VMEM((_N,), jnp.int32),
            pltpu.VMEM((_TAB,), jnp.int32),
            pltpu.VMEM((_K,), jnp.int32),
            pltpu.VMEM((16,), jnp.int32),
        ],
        compiler_params=pltpu.CompilerParams(needs_layout_passes=False),
    )(packed, ztab, fill, kvec)

    valid_sel = ranks >= 0
    top = jnp.where(valid_sel, order[jnp.where(valid_sel, ranks, 0)], -1)

    big = jnp.int32(1 << 30)
    safe_top = jnp.where(valid_sel, top, 0)
    s_sel = starts[safe_top]
    e_sel = ends[safe_top]
    m_last = ends[-1]
    a = (s_sel // 256) * m_last
    b = (s_sel % 256) * m_last + e_sel
    t_lo = (a % 256) * 256 + b
    lo = t_lo % 65536
    hi = (a // 256) + t_lo // 65536
    hi = jnp.where(valid_sel, hi, big)
    lo = jnp.where(valid_sel, lo, jnp.int32(0))
    _, _, idx = lax.sort((hi, lo, top), num_keys=2, is_stable=True)

    valid = idx >= 0
    safe = jnp.where(valid, idx, 0)
    sel_scores = jnp.where(valid, jnp.take(scores, safe), 0.0)
    return (idx, sel_scores)


# speedup vs baseline: 481.3368x; 1.3285x over previous
import jax
import jax.numpy as jnp
from jax import lax
from jax.experimental import pallas as pl
from jax.experimental.pallas import tpu as pltpu
from jax.experimental.pallas import tpu_sc as plsc

_N = 20000
_K = 4000
_CH = 125
_TAB = 100064


def _greedy_body(packed_hbm, ztab_hbm, fill_hbm, kvec_hbm, out_hbm,
                 packed_v, table_v, top_v, kv):
    cid = lax.axis_index("c")
    sid = lax.axis_index("s")

    @pl.when(jnp.logical_and(cid == 0, sid == 0))
    def _():
        pltpu.sync_copy(packed_hbm, packed_v)
        pltpu.sync_copy(ztab_hbm, table_v)
        pltpu.sync_copy(fill_hbm, top_v)
        pltpu.sync_copy(kvec_hbm, kv)
        kk = kv[...][0]
        lanes = lax.iota(jnp.int32, 16)
        d1 = lanes + 16

        def crosscheck(v0, v1, w1, lim):
            bad0 = ((lanes >= 1) & (lanes <= w1) & ((v0 >> 5) > lim - lanes)) | \
                   ((lanes < w1) & ((v0 & 31) > lanes + 1))
            bad1 = ((d1 <= w1) & ((v1 >> 5) > lim - d1)) | \
                   ((d1 < w1) & ((v1 & 31) > d1 + 1))
            return jnp.any(bad0 | bad1)

        def step(i, cnt):
            t0 = 2 * i
            swa = plsc.load_gather(
                packed_v, [jnp.full((16,), t0, jnp.int32)])[0]
            swb = plsc.load_gather(
                packed_v, [jnp.full((16,), t0 + 1, jnp.int32)])[0]
            sa = swa >> 5
            w1a = swa & 31
            lima = w1a + 1
            ea = sa + w1a
            sb = swb >> 5
            w1b = swb & 31
            limb = w1b + 1
            eb = sb + w1b

            v0a = plsc.load_gather(table_v, [sa + lanes])
            v1a = plsc.load_gather(table_v, [sa + lanes + 16])
            v0b = plsc.load_gather(table_v, [sb + lanes])
            v1b = plsc.load_gather(table_v, [sb + lanes + 16])
            tea = plsc.load_gather(
                table_v, [jnp.full((16,), ea, jnp.int32)])[0]
            teb_pre = plsc.load_gather(
                table_v, [jnp.full((16,), eb, jnp.int32)])[0]
            tsa = v0a[0]
            tsb_pre = v0b[0]

            ok_a = jnp.logical_and(
                jnp.logical_not(crosscheck(v0a, v1a, w1a, lima)), cnt < kk)
            cross_ab = ((sa < sb) & (sb <= ea) & (ea < eb)) | \
                       ((sb < sa) & (sa <= eb) & (eb < ea))
            cnt_a = cnt + jnp.where(ok_a, jnp.int32(1), jnp.int32(0))
            ok_b = jnp.logical_not(crosscheck(v0b, v1b, w1b, limb)) & \
                jnp.logical_not(ok_a & cross_ab) & (cnt_a < kk)

            addba = jnp.where(sa == ea, lima, jnp.int32(0))
            news_a = (jnp.maximum(tsa >> 5, lima) << 5) | \
                jnp.maximum(tsa & 31, addba)
            newe_a = (jnp.maximum(tea >> 5, addba) << 5) | \
                jnp.maximum(tea & 31, lima)

            tsb = jnp.where(ok_a & (sb == sa), news_a,
                            jnp.where(ok_a & (sb == ea), newe_a, tsb_pre))
            teb = jnp.where(ok_a & (eb == sa), news_a,
                            jnp.where(ok_a & (eb == ea), newe_a, teb_pre))
            addbb = jnp.where(sb == eb, limb, jnp.int32(0))
            news_b = (jnp.maximum(tsb >> 5, limb) << 5) | \
                jnp.maximum(tsb & 31, addbb)
            newe_b = (jnp.maximum(teb >> 5, addbb) << 5) | \
                jnp.maximum(teb & 31, limb)

            news_a2 = jnp.where(ok_b & (sa == sb), news_b,
                                jnp.where(ok_b & (sa == eb), newe_b, news_a))
            newe_a2 = jnp.where(ok_b & (ea == sb), news_b,
                                jnp.where(ok_b & (ea == eb), newe_b, newe_a))

            idxv = jnp.where(lanes == 0, sa,
                             jnp.where(lanes == 1, ea,
                                       jnp.where(lanes == 2, sb, eb)))
            valv = jnp.where(lanes == 0, news_a2,
                             jnp.where(lanes == 1, newe_a2,
                                       jnp.where(lanes == 2, news_b, newe_b)))
            mask = ((lanes < 2) & ok_a) | ((lanes >= 2) & (lanes < 4) & ok_b)
            plsc.store_scatter(table_v, [idxv], valv, mask=mask)
            plsc.store_scatter(
                top_v, [jnp.where(lanes == 0, cnt, cnt_a)],
                jnp.where(lanes == 0, t0, t0 + 1),
                mask=((lanes == 0) & ok_a) | ((lanes == 1) & ok_b))

            return cnt_a + jnp.where(ok_b, jnp.int32(1), jnp.int32(0))

        def chunk(ci, cnt):
            return lax.cond(
                cnt < kk,
                lambda c: lax.fori_loop(ci * _CH, (ci + 1) * _CH, step, c),
                lambda c: c,
                cnt)

        lax.fori_loop(0, (_N // 2) // _CH, chunk, jnp.int32(0))
        pltpu.sync_copy(top_v, out_hbm)


def kernel(ment_starts, ment_ends, ment_scores, k):
    starts = ment_starts.astype(jnp.int32)
    ends = ment_ends.astype(jnp.int32)
    scores = jnp.asarray(ment_scores)
    order = jnp.argsort(-scores, stable=True).astype(jnp.int32)
    ssort = starts[order]
    wsort = ends[order] - ssort
    packed = ssort * 32 + wsort

    ztab = jnp.zeros((_TAB,), jnp.int32)
    fill = jnp.full((_K,), -1, jnp.int32)
    kvec = jnp.full((16,), jnp.asarray(k, jnp.int32))

    mesh = plsc.VectorSubcoreMesh(core_axis_name="c", subcore_axis_name="s",
                                  num_cores=2, num_subcores=16)
    ranks = pl.kernel(
        _greedy_body,
        out_type=jax.ShapeDtypeStruct((_K,), jnp.int32),
        mesh=mesh,
        scratch_types=[
            pltpu.VMEM((_N,), jnp.int32),
            pltpu.VMEM((_TAB,), jnp.int32),
            pltpu.VMEM((_K,), jnp.int32),
            pltpu.VMEM((16,), jnp.int32),
        ],
        compiler_params=pltpu.CompilerParams(needs_layout_passes=False),
    )(packed, ztab, fill, kvec)

    valid_sel = ranks >= 0
    top = jnp.where(valid_sel, order[jnp.where(valid_sel, ranks, 0)], -1)

    big = jnp.int32(1 << 30)
    safe_top = jnp.where(valid_sel, top, 0)
    s_sel = starts[safe_top]
    e_sel = ends[safe_top]
    m_last = ends[-1]
    a = (s_sel // 256) * m_last
    b = (s_sel % 256) * m_last + e_sel
    t_lo = (a % 256) * 256 + b
    lo = t_lo % 65536
    hi = (a // 256) + t_lo // 65536
    hi = jnp.where(valid_sel, hi, big)
    lo = jnp.where(valid_sel, lo, jnp.int32(0))
    _, _, idx = lax.sort((hi, lo, top), num_keys=2, is_stable=True)

    valid = idx >= 0
    safe = jnp.where(valid, idx, 0)
    sel_scores = jnp.where(valid, jnp.take(scores, safe), 0.0)
    return (idx, sel_scores)


# speedup vs baseline: 600.0350x; 1.2466x over previous
import jax
import jax.numpy as jnp
from jax import lax
from jax.experimental import pallas as pl
from jax.experimental.pallas import tpu as pltpu
from jax.experimental.pallas import tpu_sc as plsc

_N = 20000
_K = 4000
_CH = 125
_TAB = 100064


def _greedy_body(packed_hbm, ztab_hbm, fill_hbm, kvec_hbm, out_hbm,
                 packed_v, table_v, top_v, kv):
    cid = lax.axis_index("c")
    sid = lax.axis_index("s")

    @pl.when(jnp.logical_and(cid == 0, sid == 0))
    def _():
        pltpu.sync_copy(packed_hbm, packed_v)
        pltpu.sync_copy(ztab_hbm, table_v)
        pltpu.sync_copy(fill_hbm, top_v)
        pltpu.sync_copy(kvec_hbm, kv)
        kk = kv[...][0]
        lanes = lax.iota(jnp.int32, 16)
        d1 = lanes + 16

        def crosscheck(v0, v1, w1, lim):
            bad0 = ((lanes >= 1) & (lanes <= w1) & ((v0 >> 5) > lim - lanes)) | \
                   ((lanes < w1) & ((v0 & 31) > lanes + 1))
            bad1 = ((d1 <= w1) & ((v1 >> 5) > lim - d1)) | \
                   ((d1 < w1) & ((v1 & 31) > d1 + 1))
            return jnp.any(bad0 | bad1)

        def step(i, cnt):
            t0 = 4 * i
            sw, s, w1, lim, e, v0, v1 = [], [], [], [], [], [], []
            for j in range(4):
                swj = plsc.load_gather(
                    packed_v, [jnp.full((16,), t0 + j, jnp.int32)])[0]
                sj = swj >> 5
                w1j = swj & 31
                sw.append(swj)
                s.append(sj)
                w1.append(w1j)
                lim.append(w1j + 1)
                e.append(sj + w1j)
            for j in range(4):
                v0.append(plsc.load_gather(table_v, [s[j] + lanes]))
                v1.append(plsc.load_gather(table_v, [s[j] + lanes + 16]))

            def cross(m, j):
                return ((s[m] < s[j]) & (s[j] <= e[m]) & (e[m] < e[j])) | \
                       ((s[j] < s[m]) & (s[m] <= e[j]) & (e[j] < e[m]))

            ok = []
            cs = [cnt]
            for j in range(4):
                okj = jnp.logical_not(crosscheck(v0[j], v1[j], w1[j], lim[j]))
                for m in range(j):
                    okj = okj & jnp.logical_not(ok[m] & cross(m, j))
                okj = okj & (cs[j] < kk)
                ok.append(okj)
                cs.append(cs[j] + jnp.where(okj, jnp.int32(1), jnp.int32(0)))

            pv = jnp.where(lanes == 0, s[0],
                           jnp.where(lanes == 1, e[0],
                                     jnp.where(lanes == 2, s[1],
                                               jnp.where(lanes == 3, e[1],
                                                         jnp.where(lanes == 4, s[2],
                                                                   jnp.where(lanes == 5, e[2],
                                                                             jnp.where(lanes == 6, s[3], e[3])))))))
            pre = plsc.load_gather(table_v, [pv])
            aval = jnp.int32(0)
            bval = jnp.int32(0)
            zero = jnp.zeros((16,), jnp.int32)
            for m in range(4):
                aval = jnp.maximum(
                    aval, jnp.where(ok[m] & (pv == s[m]), lim[m], zero))
                bval = jnp.maximum(
                    bval, jnp.where(ok[m] & (pv == e[m]), lim[m], zero))
            val = (jnp.maximum(pre >> 5, aval) << 5) | \
                jnp.maximum(pre & 31, bval)
            maskp = ((lanes < 2) & ok[0]) | \
                ((lanes >= 2) & (lanes < 4) & ok[1]) | \
                ((lanes >= 4) & (lanes < 6) & ok[2]) | \
                ((lanes >= 6) & (lanes < 8) & ok[3])
            plsc.store_scatter(table_v, [pv], val, mask=maskp)

            idxt = jnp.where(lanes == 0, cs[0],
                             jnp.where(lanes == 1, cs[1],
                                       jnp.where(lanes == 2, cs[2], cs[3])))
            maskt = ((lanes == 0) & ok[0]) | ((lanes == 1) & ok[1]) | \
                ((lanes == 2) & ok[2]) | ((lanes == 3) & ok[3])
            plsc.store_scatter(top_v, [idxt], t0 + jnp.minimum(lanes, 3),
                               mask=maskt)
            return cs[4]

        def chunk(ci, cnt):
            return lax.cond(
                cnt < kk,
                lambda c: lax.fori_loop(ci * _CH, (ci + 1) * _CH, step, c),
                lambda c: c,
                cnt)

        lax.fori_loop(0, (_N // 4) // _CH, chunk, jnp.int32(0))
        pltpu.sync_copy(top_v, out_hbm)


def kernel(ment_starts, ment_ends, ment_scores, k):
    starts = ment_starts.astype(jnp.int32)
    ends = ment_ends.astype(jnp.int32)
    scores = jnp.asarray(ment_scores)
    order = jnp.argsort(-scores, stable=True).astype(jnp.int32)
    ssort = starts[order]
    wsort = ends[order] - ssort
    packed = ssort * 32 + wsort

    ztab = jnp.zeros((_TAB,), jnp.int32)
    fill = jnp.full((_K,), -1, jnp.int32)
    kvec = jnp.full((16,), jnp.asarray(k, jnp.int32))

    mesh = plsc.VectorSubcoreMesh(core_axis_name="c", subcore_axis_name="s",
                                  num_cores=2, num_subcores=16)
    ranks = pl.kernel(
        _greedy_body,
        out_type=jax.ShapeDtypeStruct((_K,), jnp.int32),
        mesh=mesh,
        scratch_types=[
            pltpu.VMEM((_N,), jnp.int32),
            pltpu.VMEM((_TAB,), jnp.int32),
            pltpu.VMEM((_K,), jnp.int32),
            pltpu.VMEM((16,), jnp.int32),
        ],
        compiler_params=pltpu.CompilerParams(needs_layout_passes=False),
    )(packed, ztab, fill, kvec)

    valid_sel = ranks >= 0
    top = jnp.where(valid_sel, order[jnp.where(valid_sel, ranks, 0)], -1)

    big = jnp.int32(1 << 30)
    safe_top = jnp.where(valid_sel, top, 0)
    s_sel = starts[safe_top]
    e_sel = ends[safe_top]
    m_last = ends[-1]
    a = (s_sel // 256) * m_last
    b = (s_sel % 256) * m_last + e_sel
    t_lo = (a % 256) * 256 + b
    lo = t_lo % 65536
    hi = (a // 256) + t_lo // 65536
    hi = jnp.where(valid_sel, hi, big)
    lo = jnp.where(valid_sel, lo, jnp.int32(0))
    _, _, idx = lax.sort((hi, lo, top), num_keys=2, is_stable=True)

    valid = idx >= 0
    safe = jnp.where(valid, idx, 0)
    sel_scores = jnp.where(valid, jnp.take(scores, safe), 0.0)
    return (idx, sel_scores)
